# baseline (speedup 1.0000x reference)
import jax
import jax.numpy as jnp
from jax.experimental import pallas as pl

B, N, DIN, DOUT, K = 32, 256, 256, 128, 8
CPK = DOUT // K


def _body(x_ref, adj_ref, w_ref, bias_ref, out_ref):
    x = x_ref[0]
    h = jnp.dot(x, w_ref[...], preferred_element_type=jnp.float32)
    h = jnp.maximum(h + bias_ref[...], 0.0)
    outs = []
    for k in range(K):
        hk = h[:, k * CPK:(k + 1) * CPK]
        outs.append(jnp.dot(adj_ref[0, k], hk,
                            preferred_element_type=jnp.float32))
    out_ref[0] = jnp.concatenate(outs, axis=1)


def kernel(node_feats, adj, W, b):
    bias = b.reshape(1, DOUT)
    out = pl.pallas_call(
        _body,
        grid=(B,),
        in_specs=[
            pl.BlockSpec((1, N, DIN), lambda i: (i, 0, 0)),
            pl.BlockSpec((1, K, N, N), lambda i: (i, 0, 0, 0)),
            pl.BlockSpec((DIN, DOUT), lambda i: (0, 0)),
            pl.BlockSpec((1, DOUT), lambda i: (0, 0)),
        ],
        out_specs=pl.BlockSpec((1, N, DOUT), lambda i: (i, 0, 0)),
        out_shape=jax.ShapeDtypeStruct((B, N, DOUT), jnp.float32),
    )(node_feats, adj, W, bias)
    return out
